# baseline (speedup 1.0000x reference)
import functools

import jax
import jax.numpy as jnp
from jax import lax
from jax.experimental import pallas as pl
from jax.experimental.pallas import tpu as pltpu
from jax.experimental.pallas import tpu_sc as plsc

INPUT_SIZE = 64
BATCH = 16384
LANES = 16
VOCAB = 1000000
PAD_MINOR = ((VOCAB + 127) // 128) * 128

_info = plsc.get_sparse_core_info()
NC = _info.num_cores
NS = _info.num_subcores
NW = NC * NS

WIN = 512
SEG = 2048
CREDITS = 32
MCHUNK = 128

LOS = [(w * (VOCAB // NW) // 128) * 128 for w in range(NW)] + [VOCAB]
NWIN = max((LOS[w + 1] - LOS[w] + WIN - 1) // WIN for w in range(NW)) + 1

_mesh = plsc.VectorSubcoreMesh(core_axis_name="c", subcore_axis_name="s")


@functools.partial(
    pl.kernel,
    mesh=_mesh,
    compiler_params=pltpu.CompilerParams(needs_layout_passes=False),
    out_type=(
        jax.ShapeDtypeStruct((BATCH, INPUT_SIZE), jnp.float32),
        jax.ShapeDtypeStruct((8, INPUT_SIZE), jnp.float32),
    ),
    scratch_types=[
        pltpu.VMEM((128,), jnp.int32),
        pltpu.VMEM((SEG,), jnp.int32),
        pltpu.VMEM((SEG,), jnp.int32),
        pltpu.VMEM((SEG,), jnp.int32),
        pltpu.VMEM((2, INPUT_SIZE, WIN), jnp.float32),
        pltpu.VMEM((CREDITS, INPUT_SIZE), jnp.float32),
        pltpu.SemaphoreType.DMA,
        pltpu.SemaphoreType.DMA,
        pltpu.SemaphoreType.DMA,
    ],
)
def _sc_gather(inds_hbm, bs_hbm, wt_hbm, map_hbm, edges_hbm,
               g_hbm, dummy_hbm,
               edges_v, inds_v, bs_v, map_v, win_v, gcol_v,
               sem_m, sem_w, sem_g):
    wid = lax.axis_index("s") * NC + lax.axis_index("c")
    lo = pl.multiple_of((wid * (VOCAB // NW) // 128) * 128, 128)
    iota16 = lax.iota(jnp.int32, LANES)

    def pick(v, i):
        return jnp.take(v, jnp.full((LANES,), i, jnp.int32),
                        mode="fill", fill_value=0)[0]

    pltpu.sync_copy(edges_hbm, edges_v)
    ebase = pl.multiple_of((wid // 8) * 8, 8)
    evec = edges_v[pl.ds(ebase, LANES)]
    e_lo = pick(evec, wid & 7)
    e_hi = pick(evec, (wid & 7) + 1)

    for _ in range(CREDITS):
        pltpu.async_copy(gcol_v.at[0], dummy_hbm.at[0], sem_g)

    def win_start(g):
        return pl.multiple_of(
            jnp.minimum(lo + g * WIN, PAD_MINOR - WIN), 128)

    def fire_win(g):
        pltpu.async_copy(
            wt_hbm.at[pl.ds(0, INPUT_SIZE), pl.ds(win_start(g), WIN)],
            win_v.at[g & 1],
            sem_w,
        )

    def drain_win(g):
        pltpu.make_async_copy(
            wt_hbm.at[pl.ds(0, INPUT_SIZE), pl.ds(0, WIN)],
            win_v.at[g & 1],
            sem_w,
        ).wait()

    def sweep_pass(astart0):
        astart = pl.multiple_of(
            jnp.minimum(astart0, BATCH - SEG) & ~7, 8)
        pltpu.sync_copy(inds_hbm.at[pl.ds(astart, SEG)], inds_v)
        pltpu.sync_copy(bs_hbm.at[pl.ds(astart, SEG)], bs_v)

        def m_fire(k, _):
            pltpu.async_copy(
                map_hbm.at[inds_v.at[pl.ds(k * MCHUNK, MCHUNK)]],
                map_v.at[pl.ds(k * MCHUNK, MCHUNK)],
                sem_m,
            )
            return 0

        lax.fori_loop(0, SEG // MCHUNK, m_fire, 0)

        def m_drain(k, _):
            pltpu.make_async_copy(
                map_hbm.at[inds_v.at[pl.ds(0, MCHUNK)]],
                map_v.at[pl.ds(k * MCHUNK, MCHUNK)],
                sem_m,
            ).wait()
            return 0

        lax.fori_loop(0, SEG // MCHUNK, m_drain, 0)

        def extract_win(g, carry):
            ptr0, credit0 = carry
            s = win_start(g)
            c0 = ptr0 + (credit0 << 8)

            def chunk_cond(c):
                p = c & 255
                poff = pl.multiple_of(
                    jnp.minimum(p, SEG // LANES - 1) * LANES, LANES)
                first = pick(map_v[pl.ds(poff, LANES)], 0)
                return ((c >> 14) == 0) & (p < SEG // LANES) & (first < s + WIN)

            def chunk_body(c):
                p = c & 255
                credit = (c >> 8) & 63
                off = pl.multiple_of(
                    jnp.minimum(p, SEG // LANES - 1) * LANES, LANES)
                m16 = map_v[pl.ds(off, LANES)]
                b16 = bs_v[pl.ds(off, LANES)]
                inw = (m16 >= s) & (m16 < s + WIN)
                pre = plsc.all_reduce_population_count(m16 < s)[0]
                n = plsc.all_reduce_population_count(inw)[0]

                def ext(i, credit):
                    lane = pre + i
                    mrel = pick(m16, lane) - s
                    b = pick(b16, lane)
                    slot = credit & (CREDITS - 1)
                    pltpu.make_async_copy(
                        gcol_v.at[0], dummy_hbm.at[0], sem_g).wait()
                    for q in range(INPUT_SIZE // LANES):
                        vals = plsc.load_gather(
                            win_v,
                            [jnp.full((LANES,), g & 1, jnp.int32),
                             iota16 + q * LANES,
                             jnp.full((LANES,), mrel, jnp.int32)],
                        )
                        gcol_v[slot, pl.ds(q * LANES, LANES)] = vals
                    pltpu.async_copy(
                        gcol_v.at[slot], g_hbm.at[b], sem_g)
                    return (credit + 1) & 63

                credit = lax.fori_loop(0, n, ext, credit)
                consumed = plsc.all_reduce_population_count(
                    m16 < s + WIN)[0]
                full_chunk = consumed == LANES
                p = jnp.where(full_chunk, p + 1, p)
                done = jnp.where(full_chunk, 0, 1)
                return p + (credit << 8) + (done << 14)

            c1 = lax.while_loop(chunk_cond, chunk_body, c0)
            return c1 & 255, (c1 >> 8) & 63

        fire_win(0)

        def win_loop(g, carry):
            fire_win(g + 1)
            drain_win(g)
            return extract_win(g, carry)

        carry = lax.fori_loop(0, NWIN - 1, win_loop,
                              (jnp.int32(0), jnp.int32(0)))
        drain_win(NWIN - 1)
        extract_win(NWIN - 1, carry)
        return astart + SEG

    lax.while_loop(lambda a: a < e_hi, sweep_pass, e_lo & ~7)

    def g_drain(k, _):
        pltpu.make_async_copy(gcol_v.at[0], dummy_hbm.at[0], sem_g).wait()
        return 0

    lax.fori_loop(0, CREDITS, g_drain, 0)


@functools.partial(
    pl.pallas_call,
    out_shape=jax.ShapeDtypeStruct((INPUT_SIZE, BATCH), jnp.float32),
    grid=(16,),
    in_specs=[
        pl.BlockSpec((INPUT_SIZE, BATCH // 16), lambda i: (0, i)),
        pl.BlockSpec((BATCH // 16, INPUT_SIZE), lambda i: (i, 0)),
    ],
    out_specs=pl.BlockSpec((INPUT_SIZE, BATCH // 16), lambda i: (0, i)),
)
def _tc_add(xt_ref, g_ref, o_ref):
    o_ref[...] = xt_ref[...] + g_ref[...].T


def kernel(x, ind, weight, mapping):
    ind32 = ind.astype(jnp.int32)
    iota = lax.iota(jnp.int32, BATCH)
    ind_s, order = lax.sort((ind32, iota), dimension=0, num_keys=1)
    los = jnp.array(LOS, jnp.int32)
    edges = jnp.sum(ind_s[None, :] < los[:, None], axis=1,
                    dtype=jnp.int32)
    edges_padded = jnp.pad(edges, (0, 128 - edges.shape[0]),
                           constant_values=BATCH)
    g_lin, _ = _sc_gather(ind_s, order, weight.T,
                          mapping.astype(jnp.int32), edges_padded)
    return _tc_add(x.T, g_lin).T


# speedup vs baseline: 1.2797x; 1.2797x over previous
import functools

import jax
import jax.numpy as jnp
from jax import lax
from jax.experimental import pallas as pl
from jax.experimental.pallas import tpu as pltpu
from jax.experimental.pallas import tpu_sc as plsc

INPUT_SIZE = 64
BATCH = 16384
LANES = 16
VOCAB = 1000000
PAD_MINOR = ((VOCAB + 127) // 128) * 128

_info = plsc.get_sparse_core_info()
NC = _info.num_cores
NS = _info.num_subcores
NW = NC * NS

WIN = 512
SEG = 2048
CREDITS = 32
MCHUNK = 128

LOS = [(w * (VOCAB // NW) // 128) * 128 for w in range(NW)] + [VOCAB]
NWIN = max((LOS[w + 1] - LOS[w] + WIN - 1) // WIN for w in range(NW)) + 1

_mesh = plsc.VectorSubcoreMesh(core_axis_name="c", subcore_axis_name="s")


@functools.partial(
    pl.kernel,
    mesh=_mesh,
    compiler_params=pltpu.CompilerParams(needs_layout_passes=False),
    out_type=(
        jax.ShapeDtypeStruct((BATCH * INPUT_SIZE,), jnp.float32),
        jax.ShapeDtypeStruct((INPUT_SIZE,), jnp.float32),
    ),
    scratch_types=[
        pltpu.VMEM((128,), jnp.int32),
        pltpu.VMEM((SEG,), jnp.int32),
        pltpu.VMEM((SEG,), jnp.int32),
        pltpu.VMEM((SEG,), jnp.int32),
        pltpu.VMEM((2, INPUT_SIZE, WIN), jnp.float32),
        pltpu.VMEM((CREDITS * INPUT_SIZE,), jnp.float32),
        pltpu.SemaphoreType.DMA,
        pltpu.SemaphoreType.DMA,
        pltpu.SemaphoreType.DMA,
    ],
)
def _sc_gather(inds_hbm, bs_hbm, wt_hbm, map_hbm, edges_hbm,
               g_hbm, dummy_hbm,
               edges_v, inds_v, bs_v, map_v, win_v, gcol_v,
               sem_m, sem_w, sem_g):
    wid = lax.axis_index("s") * NC + lax.axis_index("c")
    lo = pl.multiple_of((wid * (VOCAB // NW) // 128) * 128, 128)
    iota16 = lax.iota(jnp.int32, LANES)

    def pick(v, i):
        return jnp.take(v, jnp.full((LANES,), i, jnp.int32),
                        mode="fill", fill_value=0)[0]

    pltpu.sync_copy(edges_hbm, edges_v)
    ebase = pl.multiple_of((wid // 8) * 8, 8)
    evec = edges_v[pl.ds(ebase, LANES)]
    e_lo = pick(evec, wid & 7)
    e_hi = pick(evec, (wid & 7) + 1)

    for _ in range(CREDITS):
        pltpu.async_copy(gcol_v.at[pl.ds(0, INPUT_SIZE)], dummy_hbm, sem_g)

    def win_start(g):
        return pl.multiple_of(
            jnp.minimum(lo + g * WIN, PAD_MINOR - WIN), 128)

    def fire_win(g):
        pltpu.async_copy(
            wt_hbm.at[pl.ds(0, INPUT_SIZE), pl.ds(win_start(g), WIN)],
            win_v.at[g & 1],
            sem_w,
        )

    def drain_win(g):
        pltpu.make_async_copy(
            wt_hbm.at[pl.ds(0, INPUT_SIZE), pl.ds(0, WIN)],
            win_v.at[g & 1],
            sem_w,
        ).wait()

    def sweep_pass(astart0):
        astart = pl.multiple_of(
            jnp.minimum(astart0, BATCH - SEG) & ~7, 8)
        pltpu.sync_copy(inds_hbm.at[pl.ds(astart, SEG)], inds_v)
        pltpu.sync_copy(bs_hbm.at[pl.ds(astart, SEG)], bs_v)

        def m_fire(k, _):
            pltpu.async_copy(
                map_hbm.at[inds_v.at[pl.ds(k * MCHUNK, MCHUNK)]],
                map_v.at[pl.ds(k * MCHUNK, MCHUNK)],
                sem_m,
            )
            return 0

        lax.fori_loop(0, SEG // MCHUNK, m_fire, 0)

        def m_drain(k, _):
            pltpu.make_async_copy(
                map_hbm.at[inds_v.at[pl.ds(0, MCHUNK)]],
                map_v.at[pl.ds(k * MCHUNK, MCHUNK)],
                sem_m,
            ).wait()
            return 0

        lax.fori_loop(0, SEG // MCHUNK, m_drain, 0)

        def extract_win(g, carry):
            ptr0, credit0 = carry
            s = win_start(g)
            c0 = ptr0 + (credit0 << 8)

            def chunk_cond(c):
                p = c & 255
                poff = pl.multiple_of(
                    jnp.minimum(p, SEG // LANES - 1) * LANES, LANES)
                first = pick(map_v[pl.ds(poff, LANES)], 0)
                return ((c >> 14) == 0) & (p < SEG // LANES) & (first < s + WIN)

            def chunk_body(c):
                p = c & 255
                credit = (c >> 8) & 63
                off = pl.multiple_of(
                    jnp.minimum(p, SEG // LANES - 1) * LANES, LANES)
                m16 = map_v[pl.ds(off, LANES)]
                b16 = bs_v[pl.ds(off, LANES)]
                inw = (m16 >= s) & (m16 < s + WIN)
                pre = plsc.all_reduce_population_count(m16 < s)[0]
                n = plsc.all_reduce_population_count(inw)[0]

                def ext(i, credit):
                    lane = pre + i
                    mrel = pick(m16, lane) - s
                    b = pick(b16, lane)
                    slot = credit & (CREDITS - 1)
                    pltpu.make_async_copy(
                        gcol_v.at[pl.ds(0, INPUT_SIZE)], dummy_hbm,
                        sem_g).wait()
                    for q in range(INPUT_SIZE // LANES):
                        vals = plsc.load_gather(
                            win_v,
                            [jnp.full((LANES,), g & 1, jnp.int32),
                             iota16 + q * LANES,
                             jnp.full((LANES,), mrel, jnp.int32)],
                        )
                        gofs = pl.multiple_of(
                            slot * INPUT_SIZE + q * LANES, LANES)
                        gcol_v[pl.ds(gofs, LANES)] = vals
                    pltpu.async_copy(
                        gcol_v.at[pl.ds(
                            pl.multiple_of(slot * INPUT_SIZE, INPUT_SIZE),
                            INPUT_SIZE)],
                        g_hbm.at[pl.ds(
                            pl.multiple_of(b * INPUT_SIZE, INPUT_SIZE),
                            INPUT_SIZE)],
                        sem_g,
                    )
                    return (credit + 1) & 63

                credit = lax.fori_loop(0, n, ext, credit)
                consumed = plsc.all_reduce_population_count(
                    m16 < s + WIN)[0]
                full_chunk = consumed == LANES
                p = jnp.where(full_chunk, p + 1, p)
                done = jnp.where(full_chunk, 0, 1)
                return p + (credit << 8) + (done << 14)

            c1 = lax.while_loop(chunk_cond, chunk_body, c0)
            return c1 & 255, (c1 >> 8) & 63

        fire_win(0)

        def win_loop(g, carry):
            fire_win(g + 1)
            drain_win(g)
            return extract_win(g, carry)

        carry = lax.fori_loop(0, NWIN - 1, win_loop,
                              (jnp.int32(0), jnp.int32(0)))
        drain_win(NWIN - 1)
        extract_win(NWIN - 1, carry)
        return astart + SEG

    lax.while_loop(lambda a: a < e_hi, sweep_pass, e_lo & ~7)

    def g_drain(k, _):
        pltpu.make_async_copy(gcol_v.at[pl.ds(0, INPUT_SIZE)], dummy_hbm,
                              sem_g).wait()
        return 0

    lax.fori_loop(0, CREDITS, g_drain, 0)


@functools.partial(
    pl.pallas_call,
    out_shape=jax.ShapeDtypeStruct((INPUT_SIZE, BATCH), jnp.float32),
    grid=(16,),
    in_specs=[
        pl.BlockSpec((INPUT_SIZE, BATCH // 16), lambda i: (0, i)),
        pl.BlockSpec((BATCH // 16, INPUT_SIZE), lambda i: (i, 0)),
    ],
    out_specs=pl.BlockSpec((INPUT_SIZE, BATCH // 16), lambda i: (0, i)),
)
def _tc_add(xt_ref, g_ref, o_ref):
    o_ref[...] = xt_ref[...] + g_ref[...].T


def kernel(x, ind, weight, mapping):
    ind32 = ind.astype(jnp.int32)
    iota = lax.iota(jnp.int32, BATCH)
    ind_s, order = lax.sort((ind32, iota), dimension=0, num_keys=1)
    los = jnp.array(LOS, jnp.int32)
    edges = jnp.sum(ind_s[None, :] < los[:, None], axis=1,
                    dtype=jnp.int32)
    edges_padded = jnp.pad(edges, (0, 128 - edges.shape[0]),
                           constant_values=BATCH)
    g_lin, _ = _sc_gather(ind_s, order, weight.T,
                          mapping.astype(jnp.int32), edges_padded)
    return _tc_add(x.T, g_lin.reshape(BATCH, INPUT_SIZE)).T
